# baseline (speedup 1.0000x reference)
import functools

import jax
import jax.numpy as jnp
from jax import lax
from jax.experimental import pallas as pl
from jax.experimental.pallas import tpu as pltpu
from jax.experimental.pallas import tpu_sc as plsc

N_USERS = 50000
N_NODES = 100000
EMB = 32
HEMB = EMB // 2
ALPHA = 1.0
BETA = 1.0
N_LAYERS = 3

NC = 2
NS = 16
CHUNK = 1024
QROWS = CHUNK // 128
HALF = N_NODES // NC
PAD_HALF = 50048
N_PAD = NC * PAD_HALF
GARBAGE = PAD_HALF
ACC_ROWS = 50176
BLEND_BLOCK = 3128


def _splat(v16, lane):
    idx = jnp.full((16, 1), lane, jnp.int32)
    dnums = lax.GatherDimensionNumbers(
        offset_dims=(), collapsed_slice_dims=(0,), start_index_map=(0,))
    return lax.gather(v16, idx, dnums, slice_sizes=(1,),
                      mode=lax.GatherScatterMode.PROMISE_IN_BOUNDS)


@functools.lru_cache(maxsize=None)
def _make_spmm(n_chunks: int):
    assert n_chunks % 2 == 0 and n_chunks >= 4
    ept128 = n_chunks * QROWS
    mesh = plsc.VectorSubcoreMesh(core_axis_name="c", subcore_axis_name="s")

    buf_types = [
        pltpu.VMEM((QROWS, 128), jnp.int32),
        pltpu.VMEM((QROWS, 128), jnp.int32),
        pltpu.VMEM((QROWS, 128), jnp.float32),
        pltpu.VMEM((QROWS, 128), jnp.int32),
        pltpu.VMEM((CHUNK, HEMB), jnp.float32),
        pltpu.SemaphoreType.DMA,
        pltpu.SemaphoreType.DMA,
        pltpu.SemaphoreType.DMA,
    ]

    @functools.partial(
        pl.kernel,
        out_type=(jax.ShapeDtypeStruct((N_PAD, HEMB), jnp.float32),
                  jax.ShapeDtypeStruct((N_PAD, HEMB), jnp.float32)),
        mesh=mesh,
        compiler_params=pltpu.CompilerParams(use_tc_tiling_on_sc=False),
        scratch_types=buf_types + buf_types + [
            pltpu.VMEM_SHARED((ACC_ROWS, HEMB), jnp.float32),
        ],
    )
    def spmm(cols_hbm, rows_hbm, vals_hbm, x_lo, x_hi, out_lo, out_hi,
             *refs):
        A, B, acc = refs[0:8], refs[8:16], refs[16]
        c = lax.axis_index("c")
        s = lax.axis_index("s")
        base_out = c * HALF
        astripe = s * (ACC_ROWS // NS)
        ostripe = s * (PAD_HALF // NS)

        def b128(j):
            return s * ept128 + j * QROWS

        def idx_issue(j, P):
            pltpu.async_copy(cols_hbm.at[pl.ds(b128(j), QROWS)], P[0], P[5])
            pltpu.async_copy(rows_hbm.at[pl.ds(b128(j), QROWS)], P[1], P[5])
            pltpu.async_copy(vals_hbm.at[pl.ds(b128(j), QROWS)], P[2], P[5])

        def idx_wait(P):
            pltpu.make_async_copy(cols_hbm.at[pl.ds(0, QROWS)], P[0], P[5]).wait()
            pltpu.make_async_copy(rows_hbm.at[pl.ds(0, QROWS)], P[1], P[5]).wait()
            pltpu.make_async_copy(vals_hbm.at[pl.ds(0, QROWS)], P[2], P[5]).wait()

        def gth_issue(xh, P):
            for q in range(QROWS):
                pltpu.async_copy(xh.at[P[0].at[q]],
                                 P[4].at[pl.ds(q * 128, 128)], P[6])

        def gth_wait(xh, P):
            for q in range(QROWS):
                pltpu.make_async_copy(xh.at[P[0].at[q]],
                                      P[4].at[pl.ds(q * 128, 128)], P[6]).wait()

        def scat_issue(P):
            for q in range(QROWS):
                pltpu.async_copy(P[4].at[pl.ds(q * 128, 128)],
                                 acc.at[P[3].at[q]], P[7], add=True)

        def scat_wait(P):
            for q in range(QROWS):
                pltpu.make_async_copy(P[4].at[pl.ds(q * 128, 128)],
                                      acc.at[P[3].at[q]], P[7]).wait()

        def compute(P):
            def gbody(g, carry):
                q = g // 8
                lo = (g % 8) * 16
                r16 = P[1][q, pl.ds(lo, 16)]
                loc = r16 - base_out
                ok = (loc >= 0) & (loc < HALF)
                P[3][q, pl.ds(lo, 16)] = jnp.where(ok, loc, GARBAGE)
                v16 = P[2][q, pl.ds(lo, 16)]
                e0 = g * 16
                for lane in range(16):
                    sp = _splat(v16, lane)
                    P[4][e0 + lane, pl.ds(0, 16)] = (
                        P[4][e0 + lane, pl.ds(0, 16)] * sp)
                return carry
            lax.fori_loop(0, CHUNK // 16, gbody, 0)

        def section(j, xh, P, Q, do_scwait, do_next, do_idx2):
            if do_next:
                idx_wait(Q)
            if do_scwait:
                scat_wait(Q)
            if do_next:
                gth_issue(xh, Q)
            gth_wait(xh, P)
            compute(P)
            scat_issue(P)
            if do_idx2:
                idx_issue(j + 2, P)

        for p, (x_hbm, out_hbm) in enumerate(((x_lo, out_lo), (x_hi, out_hi))):
            def zrow(e, carry):
                A[4][e, pl.ds(0, 16)] = jnp.zeros((16,), jnp.float32)
                return carry
            lax.fori_loop(0, CHUNK, zrow, 0)
            for k in range(3):
                pltpu.sync_copy(A[4], acc.at[pl.ds(astripe + k * CHUNK, CHUNK)])
            pltpu.sync_copy(A[4].at[pl.ds(0, 64)],
                            acc.at[pl.ds(astripe + 3 * CHUNK, 64)])
            plsc.subcore_barrier()

            idx_issue(0, A)
            idx_wait(A)
            gth_issue(x_hbm, A)
            idx_issue(1, B)
            section(0, x_hbm, A, B, False, True, True)
            section(1, x_hbm, B, A, True, True, True)

            def pair(t, carry):
                j = 2 * t
                section(j, x_hbm, A, B, True, True, True)
                section(j + 1, x_hbm, B, A, True, True, True)
                return carry
            lax.fori_loop(1, n_chunks // 2 - 1, pair, 0)

            section(n_chunks - 2, x_hbm, A, B, True, True, False)
            section(n_chunks - 1, x_hbm, B, A, True, False, False)
            scat_wait(B)
            plsc.subcore_barrier()

            pltpu.sync_copy(acc.at[pl.ds(ostripe, PAD_HALF // NS)],
                            out_hbm.at[pl.ds(c * PAD_HALF + ostripe, PAD_HALF // NS)])
            if p == 0:
                plsc.subcore_barrier()

    return spmm


def _blend_body(el_ref, eh_ref, nl_ref, nh_ref, ol_ref, oh_ref):
    el = el_ref[...]
    eh = eh_ref[...]
    nl = nl_ref[...]
    nh = nh_ref[...]
    dl = el - nl + 1e-6
    dh = eh - nh + 1e-6
    ss = jnp.sum(dl * dl, axis=1, keepdims=True) + jnp.sum(dh * dh, axis=1, keepdims=True)
    os_score = jnp.sqrt(ss) * BETA
    d_new = ALPHA * jnp.log1p(os_score)
    inv = 1.0 / (1.0 + d_new)
    ol_ref[...] = (el + d_new * nl) * inv
    oh_ref[...] = (eh + d_new * nh) * inv


_tc_blend = pl.pallas_call(
    _blend_body,
    grid=(N_PAD // BLEND_BLOCK,),
    in_specs=[pl.BlockSpec((BLEND_BLOCK, HEMB), lambda i: (i, 0))] * 4,
    out_specs=[pl.BlockSpec((BLEND_BLOCK, HEMB), lambda i: (i, 0))] * 2,
    out_shape=(jax.ShapeDtypeStruct((N_PAD, HEMB), jnp.float32),
               jax.ShapeDtypeStruct((N_PAD, HEMB), jnp.float32)),
)


def kernel(user_emb, item_emb, adj_vals, adj_rows, adj_cols):
    zpad = jnp.zeros((PAD_HALF - HALF, HEMB), jnp.float32)
    ego_lo = jnp.concatenate(
        [user_emb[:, :HEMB], zpad, item_emb[:, :HEMB], zpad], axis=0)
    ego_hi = jnp.concatenate(
        [user_emb[:, HEMB:], zpad, item_emb[:, HEMB:], zpad], axis=0)

    n_edges = adj_rows.shape[0]
    per_tile = NS * CHUNK
    n_chunks = max(4, 2 * (-(-n_edges // (2 * per_tile))))
    e_pad = n_chunks * per_tile
    pad = e_pad - n_edges
    cols_adj = jnp.where(adj_cols < HALF, adj_cols, adj_cols + (PAD_HALF - HALF))
    rows_p = jnp.concatenate(
        [adj_rows, jnp.full((pad,), N_NODES, jnp.int32)]).reshape(e_pad // 128, 128)
    cols_p = jnp.concatenate(
        [cols_adj, jnp.zeros((pad,), jnp.int32)]).reshape(e_pad // 128, 128)
    vals_p = jnp.concatenate(
        [adj_vals, jnp.zeros((pad,), jnp.float32)]).reshape(e_pad // 128, 128)

    spmm = _make_spmm(n_chunks)
    layer_los, layer_his = [], []
    for _ in range(N_LAYERS):
        new_lo, new_hi = spmm(cols_p, rows_p, vals_p, ego_lo, ego_hi)
        ego_lo, ego_hi = _tc_blend(ego_lo, ego_hi, new_lo, new_hi)
        layer_los.append(ego_lo)
        layer_his.append(ego_hi)
    embs = jnp.concatenate([jnp.stack(layer_los, axis=1),
                            jnp.stack(layer_his, axis=1)], axis=2)
    ego = jnp.concatenate([ego_lo, ego_hi], axis=1)
    item_lo = PAD_HALF
    item_hi = PAD_HALF + (N_NODES - N_USERS)
    return (ego[:N_USERS], ego[item_lo:item_hi],
            embs[:N_USERS], embs[item_lo:item_hi])


# speedup vs baseline: 2.6840x; 2.6840x over previous
import functools

import jax
import jax.numpy as jnp
from jax import lax
from jax.experimental import pallas as pl
from jax.experimental.pallas import tpu as pltpu
from jax.experimental.pallas import tpu_sc as plsc

N_USERS = 50000
N_NODES = 100000
EMB = 32
HEMB = EMB // 2
ALPHA = 1.0
BETA = 1.0
N_LAYERS = 3

NC = 2
NS = 16
CHUNK = 1024
QROWS = CHUNK // 128
HALF = N_NODES // NC
PAD_HALF = 50048
N_PAD = NC * PAD_HALF
GARBAGE = PAD_HALF
ACC_ROWS = 50176
BLEND_BLOCK = 3128


def _splat(v16, lane):
    idx = jnp.full((16, 1), lane, jnp.int32)
    dnums = lax.GatherDimensionNumbers(
        offset_dims=(), collapsed_slice_dims=(0,), start_index_map=(0,))
    return lax.gather(v16, idx, dnums, slice_sizes=(1,),
                      mode=lax.GatherScatterMode.PROMISE_IN_BOUNDS)


@functools.lru_cache(maxsize=None)
def _make_spmm(n_chunks: int):
    assert n_chunks % 2 == 0 and n_chunks >= 4
    ept128 = n_chunks * QROWS
    mesh = plsc.VectorSubcoreMesh(core_axis_name="c", subcore_axis_name="s")

    buf_types = [
        pltpu.VMEM((QROWS, 128), jnp.int32),
        pltpu.VMEM((QROWS, 128), jnp.int32),
        pltpu.VMEM((QROWS, 128), jnp.float32),
        pltpu.VMEM((QROWS, 128), jnp.int32),
        pltpu.VMEM((CHUNK, HEMB), jnp.float32),
        pltpu.SemaphoreType.DMA,
        pltpu.SemaphoreType.DMA,
        pltpu.SemaphoreType.DMA,
    ]

    @functools.partial(
        pl.kernel,
        out_type=(jax.ShapeDtypeStruct((N_PAD, HEMB), jnp.float32),
                  jax.ShapeDtypeStruct((N_PAD, HEMB), jnp.float32)),
        mesh=mesh,
        compiler_params=pltpu.CompilerParams(use_tc_tiling_on_sc=False),
        scratch_types=buf_types + buf_types + [
            pltpu.VMEM_SHARED((ACC_ROWS, HEMB), jnp.float32),
        ],
    )
    def spmm(cols_hbm, rows_hbm, vals_hbm, x_lo, x_hi, out_lo, out_hi,
             *refs):
        A, B, acc = refs[0:8], refs[8:16], refs[16]
        c = lax.axis_index("c")
        s = lax.axis_index("s")
        base_out = c * HALF
        astripe = s * (ACC_ROWS // NS)
        ostripe = s * (PAD_HALF // NS)

        def b128(j):
            return s * ept128 + j * QROWS

        def idx_issue(j, P):
            pltpu.async_copy(cols_hbm.at[pl.ds(b128(j), QROWS)], P[0], P[5])
            pltpu.async_copy(rows_hbm.at[pl.ds(b128(j), QROWS)], P[1], P[5])
            pltpu.async_copy(vals_hbm.at[pl.ds(b128(j), QROWS)], P[2], P[5])

        def idx_wait(P):
            pltpu.make_async_copy(cols_hbm.at[pl.ds(0, QROWS)], P[0], P[5]).wait()
            pltpu.make_async_copy(rows_hbm.at[pl.ds(0, QROWS)], P[1], P[5]).wait()
            pltpu.make_async_copy(vals_hbm.at[pl.ds(0, QROWS)], P[2], P[5]).wait()

        def gth_issue(xh, P):
            for q in range(QROWS):
                pltpu.async_copy(xh.at[P[0].at[q]],
                                 P[4].at[pl.ds(q * 128, 128)], P[6])

        def gth_wait(xh, P):
            for q in range(QROWS):
                pltpu.make_async_copy(xh.at[P[0].at[q]],
                                      P[4].at[pl.ds(q * 128, 128)], P[6]).wait()

        def scat_issue(P):
            for q in range(QROWS):
                pltpu.async_copy(P[4].at[pl.ds(q * 128, 128)],
                                 acc.at[P[3].at[q]], P[7], add=True)

        def scat_wait(P):
            for q in range(QROWS):
                pltpu.make_async_copy(P[4].at[pl.ds(q * 128, 128)],
                                      acc.at[P[3].at[q]], P[7]).wait()

        def compute(P):
            def gbody(g, carry):
                q = g // 8
                lo = (g % 8) * 16
                r16 = P[1][q, pl.ds(lo, 16)]
                loc = r16 - base_out
                ok = (loc >= 0) & (loc < HALF)
                garb = GARBAGE + lo + lax.iota(jnp.int32, 16)
                P[3][q, pl.ds(lo, 16)] = jnp.where(ok, loc, garb)
                v16 = P[2][q, pl.ds(lo, 16)]
                e0 = g * 16
                for lane in range(16):
                    sp = _splat(v16, lane)
                    P[4][e0 + lane, pl.ds(0, 16)] = (
                        P[4][e0 + lane, pl.ds(0, 16)] * sp)
                return carry
            lax.fori_loop(0, CHUNK // 16, gbody, 0)

        def section(j, xh, P, Q, do_scwait, do_next, do_idx2):
            if do_next:
                idx_wait(Q)
            if do_scwait:
                scat_wait(Q)
            if do_next:
                gth_issue(xh, Q)
            gth_wait(xh, P)
            compute(P)
            scat_issue(P)
            if do_idx2:
                idx_issue(j + 2, P)

        for p, (x_hbm, out_hbm) in enumerate(((x_lo, out_lo), (x_hi, out_hi))):
            def zrow(e, carry):
                A[4][e, pl.ds(0, 16)] = jnp.zeros((16,), jnp.float32)
                return carry
            lax.fori_loop(0, CHUNK, zrow, 0)
            for k in range(3):
                pltpu.sync_copy(A[4], acc.at[pl.ds(astripe + k * CHUNK, CHUNK)])
            pltpu.sync_copy(A[4].at[pl.ds(0, 64)],
                            acc.at[pl.ds(astripe + 3 * CHUNK, 64)])
            plsc.subcore_barrier()

            idx_issue(0, A)
            idx_wait(A)
            gth_issue(x_hbm, A)
            idx_issue(1, B)
            section(0, x_hbm, A, B, False, True, True)
            section(1, x_hbm, B, A, True, True, True)

            def pair(t, carry):
                j = 2 * t
                section(j, x_hbm, A, B, True, True, True)
                section(j + 1, x_hbm, B, A, True, True, True)
                return carry
            lax.fori_loop(1, n_chunks // 2 - 1, pair, 0)

            section(n_chunks - 2, x_hbm, A, B, True, True, False)
            section(n_chunks - 1, x_hbm, B, A, True, False, False)
            scat_wait(B)
            plsc.subcore_barrier()

            pltpu.sync_copy(acc.at[pl.ds(ostripe, PAD_HALF // NS)],
                            out_hbm.at[pl.ds(c * PAD_HALF + ostripe, PAD_HALF // NS)])
            if p == 0:
                plsc.subcore_barrier()

    return spmm


def _blend_body(el_ref, eh_ref, nl_ref, nh_ref, ol_ref, oh_ref):
    el = el_ref[...]
    eh = eh_ref[...]
    nl = nl_ref[...]
    nh = nh_ref[...]
    dl = el - nl + 1e-6
    dh = eh - nh + 1e-6
    ss = jnp.sum(dl * dl, axis=1, keepdims=True) + jnp.sum(dh * dh, axis=1, keepdims=True)
    os_score = jnp.sqrt(ss) * BETA
    d_new = ALPHA * jnp.log1p(os_score)
    inv = 1.0 / (1.0 + d_new)
    ol_ref[...] = (el + d_new * nl) * inv
    oh_ref[...] = (eh + d_new * nh) * inv


_tc_blend = pl.pallas_call(
    _blend_body,
    grid=(N_PAD // BLEND_BLOCK,),
    in_specs=[pl.BlockSpec((BLEND_BLOCK, HEMB), lambda i: (i, 0))] * 4,
    out_specs=[pl.BlockSpec((BLEND_BLOCK, HEMB), lambda i: (i, 0))] * 2,
    out_shape=(jax.ShapeDtypeStruct((N_PAD, HEMB), jnp.float32),
               jax.ShapeDtypeStruct((N_PAD, HEMB), jnp.float32)),
)


def kernel(user_emb, item_emb, adj_vals, adj_rows, adj_cols):
    zpad = jnp.zeros((PAD_HALF - HALF, HEMB), jnp.float32)
    ego_lo = jnp.concatenate(
        [user_emb[:, :HEMB], zpad, item_emb[:, :HEMB], zpad], axis=0)
    ego_hi = jnp.concatenate(
        [user_emb[:, HEMB:], zpad, item_emb[:, HEMB:], zpad], axis=0)

    n_edges = adj_rows.shape[0]
    per_tile = NS * CHUNK
    n_chunks = max(4, 2 * (-(-n_edges // (2 * per_tile))))
    e_pad = n_chunks * per_tile
    pad = e_pad - n_edges
    cols_adj = jnp.where(adj_cols < HALF, adj_cols, adj_cols + (PAD_HALF - HALF))
    rows_p = jnp.concatenate(
        [adj_rows, jnp.full((pad,), N_NODES, jnp.int32)]).reshape(e_pad // 128, 128)
    cols_p = jnp.concatenate(
        [cols_adj, jnp.zeros((pad,), jnp.int32)]).reshape(e_pad // 128, 128)
    vals_p = jnp.concatenate(
        [adj_vals, jnp.zeros((pad,), jnp.float32)]).reshape(e_pad // 128, 128)

    spmm = _make_spmm(n_chunks)
    layer_los, layer_his = [], []
    for _ in range(N_LAYERS):
        new_lo, new_hi = spmm(cols_p, rows_p, vals_p, ego_lo, ego_hi)
        ego_lo, ego_hi = _tc_blend(ego_lo, ego_hi, new_lo, new_hi)
        layer_los.append(ego_lo)
        layer_his.append(ego_hi)
    embs = jnp.concatenate([jnp.stack(layer_los, axis=1),
                            jnp.stack(layer_his, axis=1)], axis=2)
    ego = jnp.concatenate([ego_lo, ego_hi], axis=1)
    item_lo = PAD_HALF
    item_hi = PAD_HALF + (N_NODES - N_USERS)
    return (ego[:N_USERS], ego[item_lo:item_hi],
            embs[:N_USERS], embs[item_lo:item_hi])
